# baseline (speedup 1.0000x reference)
import functools

import jax
import jax.numpy as jnp
from jax import lax
from jax.experimental import pallas as pl
from jax.experimental.pallas import tpu as pltpu
from jax.experimental.pallas import tpu_sc as plsc

N = 10000
D = 128
H = 128
L = 32
C = 16
E = 320000

NC = 2
NS = 16
NW = NC * NS
LANE = 128
CPW = (-(-E // (NW * LANE)) + 7) // 8 * 8
E_PAD = NW * LANE * CPW
GRP = 8
CPW0 = 160
CPW1 = 2 * CPW - CPW0
N_PAD = 10112
RPT = N_PAD // NS
AW = 144

_f32 = jnp.float32


def _make_spmm(w: int):
  mesh = plsc.VectorSubcoreMesh(core_axis_name="c", subcore_axis_name="s",
                                num_cores=NC, num_subcores=NS)

  out_type = jax.ShapeDtypeStruct((NC, N_PAD, w), _f32)
  scratch = [
      pltpu.VMEM((GRP, LANE), jnp.int32),
      pltpu.VMEM((GRP, LANE), jnp.int32),
      pltpu.VMEM((LANE, w), _f32),
      pltpu.VMEM((LANE, w), _f32),
      pltpu.VMEM_SHARED((N_PAD, w), _f32),
      pltpu.SemaphoreType.DMA,
      pltpu.SemaphoreType.DMA,
      pltpu.SemaphoreType.DMA,
      pltpu.SemaphoreType.DMA,
  ]

  def body(g, srcb, dstb, zacc, acc_out, src_v, dst_v, rows_a, rows_b,
           acc_s, sga, sgb, ssa, ssb):
    c = lax.axis_index("c")
    s = lax.axis_index("s")
    base_w = s * (CPW0 + CPW1) + c * CPW0
    ngrp = jnp.where(c == 0, CPW0 // GRP, CPW1 // GRP)

    pltpu.sync_copy(zacc, acc_s.at[pl.ds(s * RPT, RPT)])
    plsc.subcore_barrier()

    rows = (rows_a, rows_b)
    sg = (sga, sgb)
    ss = (ssa, ssb)

    def group(gi, carry):
      base = base_w + gi * GRP
      pltpu.sync_copy(srcb.at[pl.ds(base, GRP)], src_v)
      pltpu.sync_copy(dstb.at[pl.ds(base, GRP)], dst_v)

      gd = [None, None]
      sd = [None, None]
      gd[0] = pltpu.async_copy(g.at[src_v.at[0]], rows[0], sg[0])
      for j in range(1, GRP + 1):
        p, q = (j - 1) % 2, j % 2
        gd[p].wait()
        if j < GRP:
          if sd[q] is not None:
            sd[q].wait()
          gd[q] = pltpu.async_copy(g.at[src_v.at[j]], rows[q], sg[q])
        sd[p] = pltpu.async_copy(rows[p], acc_s.at[dst_v.at[j - 1]],
                                 ss[p], add=True)
      sd[0].wait()
      sd[1].wait()
      return carry

    lax.fori_loop(0, ngrp, group, 0)
    plsc.subcore_barrier()

    r0 = s * RPT
    pltpu.sync_copy(acc_s.at[pl.ds(r0, RPT)],
                    acc_out.at[c].at[pl.ds(r0, RPT)])

  return pl.kernel(body, out_type=out_type, mesh=mesh, scratch_types=scratch,
                   compiler_params=pltpu.CompilerParams(
                       use_tc_tiling_on_sc=False))


@functools.lru_cache(maxsize=None)
def _get_spmm(w: int):
  return _make_spmm(w)


_BM = 632
_GRID = (N_PAD // _BM,)


def _row_spec(w):
  return pl.BlockSpec((_BM, w), lambda i: (i, 0))


def _full_spec(r, w):
  return pl.BlockSpec((r, w), lambda i: (0, 0))


def _tc1_body(x, ws1, wn1, b1, wfcm, bfcm, wmum, bmum, s1, g1, zm):
  h0 = jnp.log(x[...] + 1.0)
  s1[...] = jnp.dot(h0, ws1[...], preferred_element_type=_f32) + b1[...]
  g1[:, :H] = jnp.dot(h0, wn1[...], preferred_element_type=_f32)
  g1[:, H:] = jnp.ones((_BM, AW - H), _f32)
  hm = jnp.maximum(
      jnp.dot(h0, wfcm[...], preferred_element_type=_f32) + bfcm[...], 0.0)
  zm[...] = jnp.dot(hm, wmum[...], preferred_element_type=_f32) + bmum[...]


_tc1 = pl.pallas_call(
    _tc1_body,
    grid=_GRID,
    in_specs=[_row_spec(D), _full_spec(D, H), _full_spec(D, H),
              _full_spec(1, H), _full_spec(D, H), _full_spec(1, H),
              _full_spec(H, L), _full_spec(1, L)],
    out_specs=[_row_spec(H), _row_spec(AW), _row_spec(L)],
    out_shape=[jax.ShapeDtypeStruct((N_PAD, H), _f32),
               jax.ShapeDtypeStruct((N_PAD, AW), _f32),
               jax.ShapeDtypeStruct((N_PAD, L), _f32)],
)


def _combine(s_ref, a0, a1, invd):
  t = s_ref[...] + (a0 + a1) * invd
  t = jnp.maximum(t, 0.0)
  nrm = jnp.sqrt(jnp.sum(t * t, axis=-1, keepdims=True))
  return t / jnp.maximum(nrm, 1e-12)


def _invdeg(a0, a1):
  return 1.0 / jnp.maximum(a0[:, H:H + 1] + a1[:, H:H + 1], 1.0)


def _tc2_body(s1, a0, a1, ws2, wn2, b2, s2, g2):
  invd = _invdeg(a0[...], a1[...])
  h1 = _combine(s1, a0[:, :H], a1[:, :H], invd)
  s2[...] = jnp.dot(h1, ws2[...], preferred_element_type=_f32) + b2[...]
  g2[...] = jnp.dot(h1, wn2[...], preferred_element_type=_f32)


_tc2 = pl.pallas_call(
    _tc2_body,
    grid=_GRID,
    in_specs=[_row_spec(H), _row_spec(AW), _row_spec(AW),
              _full_spec(H, H), _full_spec(H, H), _full_spec(1, H)],
    out_specs=[_row_spec(H), _row_spec(H)],
    out_shape=[jax.ShapeDtypeStruct((N_PAD, H), _f32),
               jax.ShapeDtypeStruct((N_PAD, H), _f32)],
)


def _tc3_body(s2, a0, a1, d0, d1, zm,
              wmu, bmu, wdec, bdec, wsc, bsc, wr, br, wdo, bdo,
              scale, r_out, do_out):
  invd = _invdeg(d0[...], d1[...])
  h2 = _combine(s2, a0[...], a1[...], invd)
  z_loc = jnp.dot(h2, wmu[...], preferred_element_type=_f32) + bmu[...]
  hz = z_loc * zm[...]
  px = jnp.maximum(
      jnp.dot(hz, wdec[...], preferred_element_type=_f32) + bdec[...], 0.0)
  logits = jnp.dot(px, wsc[...], preferred_element_type=_f32) + bsc[...]
  m = jnp.max(logits, axis=-1, keepdims=True)
  e = jnp.exp(logits - m)
  scale[...] = e / jnp.sum(e, axis=-1, keepdims=True)
  r_out[...] = jnp.dot(px, wr[...], preferred_element_type=_f32) + br[...]
  do_out[...] = jnp.dot(px, wdo[...], preferred_element_type=_f32) + bdo[...]


_tc3 = pl.pallas_call(
    _tc3_body,
    grid=_GRID,
    in_specs=[_row_spec(H), _row_spec(H), _row_spec(H),
              _row_spec(AW), _row_spec(AW), _row_spec(L),
              _full_spec(H, L), _full_spec(1, L),
              _full_spec(L, H), _full_spec(1, H),
              _full_spec(H, C), _full_spec(1, C),
              _full_spec(H, D), _full_spec(1, D),
              _full_spec(H, D), _full_spec(1, D)],
    out_specs=[_row_spec(C), _row_spec(D), _row_spec(D)],
    out_shape=[jax.ShapeDtypeStruct((N_PAD, C), _f32),
               jax.ShapeDtypeStruct((N_PAD, D), _f32),
               jax.ShapeDtypeStruct((N_PAD, D), _f32)],
)


def kernel(x, edge_index, W_self1, W_neigh1, b1, W_self2, W_neigh2, b2,
           W_mu, b_mu, W_fcm, b_fcm, W_mum, b_mum, W_dec, b_dec,
           W_sc, b_sc, W_r, b_r, W_do, b_do):
  src = edge_index[0].astype(jnp.int32)
  dst = edge_index[1].astype(jnp.int32)
  pad = E_PAD - E
  srcb = jnp.concatenate([src, jnp.zeros((pad,), jnp.int32)])
  dstb = jnp.concatenate([dst, jnp.full((pad,), N, jnp.int32)])
  srcb = srcb.reshape(E_PAD // LANE, LANE)
  dstb = dstb.reshape(E_PAD // LANE, LANE)

  xp = jnp.zeros((N_PAD, D), _f32).at[:N].set(x)
  zacc_a = jnp.zeros((RPT, AW), _f32)
  zacc_h = jnp.zeros((RPT, H), _f32)
  r1 = lambda b: b.reshape(1, -1)

  s1, g1, zm = _tc1(xp, W_self1, W_neigh1, r1(b1), W_fcm, r1(b_fcm),
                    W_mum, r1(b_mum))
  acc1 = _get_spmm(AW)(g1, srcb, dstb, zacc_a)
  s2, g2 = _tc2(s1, acc1[0], acc1[1], W_self2, W_neigh2, r1(b2))
  acc2 = _get_spmm(H)(g2, srcb, dstb, zacc_h)
  px_scale, px_r, px_dropout = _tc3(
      s2, acc2[0], acc2[1], acc1[0], acc1[1], zm,
      W_mu, r1(b_mu), W_dec, r1(b_dec), W_sc, r1(b_sc),
      W_r, r1(b_r), W_do, r1(b_do))
  return (px_scale[:N], px_r[:N], px_dropout[:N])


# speedup vs baseline: 1.3373x; 1.3373x over previous
import functools

import jax
import jax.numpy as jnp
from jax import lax
from jax.experimental import pallas as pl
from jax.experimental.pallas import tpu as pltpu
from jax.experimental.pallas import tpu_sc as plsc

N = 10000
D = 128
H = 128
L = 32
C = 16
E = 320000

NC = 2
NS = 16
NW = NC * NS
LANE = 128
CPW = (-(-E // (NW * LANE)) + 7) // 8 * 8
E_PAD = NW * LANE * CPW
GRP = 8
CPW0 = 120
CPW1 = 2 * CPW - CPW0
N_PAD = 10112
RPT = N_PAD // NS
AW = 144

_f32 = jnp.float32


def _make_spmm(w: int):
  mesh = plsc.VectorSubcoreMesh(core_axis_name="c", subcore_axis_name="s",
                                num_cores=NC, num_subcores=NS)

  out_type = jax.ShapeDtypeStruct((NC, N_PAD, w), _f32)
  scratch = [
      pltpu.VMEM((GRP, LANE), jnp.int32),
      pltpu.VMEM((GRP, LANE), jnp.int32),
      pltpu.VMEM((LANE, w), _f32),
      pltpu.VMEM((LANE, w), _f32),
      pltpu.VMEM((8, w), _f32),
      pltpu.VMEM_SHARED((N_PAD, w), _f32),
      pltpu.SemaphoreType.DMA,
      pltpu.SemaphoreType.DMA,
      pltpu.SemaphoreType.DMA,
      pltpu.SemaphoreType.DMA,
  ]

  def body(g, srcb, dstb, acc_out, src_v, dst_v, rows_a, rows_b, zb,

           acc_s, sga, sgb, ssa, ssb):
    c = lax.axis_index("c")
    s = lax.axis_index("s")
    base_w = s * (CPW0 + CPW1) + c * CPW0
    ngrp = jnp.where(c == 0, CPW0 // GRP, CPW1 // GRP)

    zero16 = jnp.zeros((16,), _f32)
    for r in range(8):
      for q in range(w // 16):
        zb[r, pl.ds(q * 16, 16)] = zero16

    def zcp(k, carry):
      pltpu.sync_copy(zb, acc_s.at[pl.ds(s * RPT + k * 8, 8)])
      return carry

    lax.fori_loop(0, RPT // 8, zcp, 0)
    plsc.subcore_barrier()

    rows = (rows_a, rows_b)
    sg = (sga, sgb)
    ss = (ssa, ssb)

    def group(gi, carry):
      base = base_w + gi * GRP
      pltpu.sync_copy(srcb.at[pl.ds(base, GRP)], src_v)
      pltpu.sync_copy(dstb.at[pl.ds(base, GRP)], dst_v)

      gd = [None, None]
      sd = [None, None]
      gd[0] = pltpu.async_copy(g.at[src_v.at[0]], rows[0], sg[0])
      for j in range(1, GRP + 1):
        p, q = (j - 1) % 2, j % 2
        gd[p].wait()
        if j < GRP:
          if sd[q] is not None:
            sd[q].wait()
          gd[q] = pltpu.async_copy(g.at[src_v.at[j]], rows[q], sg[q])
        sd[p] = pltpu.async_copy(rows[p], acc_s.at[dst_v.at[j - 1]],
                                 ss[p], add=True)
      sd[0].wait()
      sd[1].wait()
      return carry

    lax.fori_loop(0, ngrp, group, 0)
    plsc.subcore_barrier()

    r0 = s * RPT
    pltpu.sync_copy(acc_s.at[pl.ds(r0, RPT)],
                    acc_out.at[c].at[pl.ds(r0, RPT)])

  return pl.kernel(body, out_type=out_type, mesh=mesh, scratch_types=scratch,
                   compiler_params=pltpu.CompilerParams(
                       use_tc_tiling_on_sc=False))


@functools.lru_cache(maxsize=None)
def _get_spmm(w: int):
  return _make_spmm(w)


_BM = 632
_GRID = (N_PAD // _BM,)


def _row_spec(w):
  return pl.BlockSpec((_BM, w), lambda i: (i, 0))


def _full_spec(r, w):
  return pl.BlockSpec((r, w), lambda i: (0, 0))


def _tc1_body(x, ws1, wn1, b1, wfcm, bfcm, wmum, bmum, s1, g1, zm):
  h0 = jnp.log(x[...] + 1.0)
  s1[...] = jnp.dot(h0, ws1[...], preferred_element_type=_f32) + b1[...]
  g1[:, :H] = jnp.dot(h0, wn1[...], preferred_element_type=_f32)
  g1[:, H:] = jnp.ones((_BM, AW - H), _f32)
  hm = jnp.maximum(
      jnp.dot(h0, wfcm[...], preferred_element_type=_f32) + bfcm[...], 0.0)
  zm[...] = jnp.dot(hm, wmum[...], preferred_element_type=_f32) + bmum[...]


_tc1 = pl.pallas_call(
    _tc1_body,
    grid=_GRID,
    in_specs=[_row_spec(D), _full_spec(D, H), _full_spec(D, H),
              _full_spec(1, H), _full_spec(D, H), _full_spec(1, H),
              _full_spec(H, L), _full_spec(1, L)],
    out_specs=[_row_spec(H), _row_spec(AW), _row_spec(L)],
    out_shape=[jax.ShapeDtypeStruct((N_PAD, H), _f32),
               jax.ShapeDtypeStruct((N_PAD, AW), _f32),
               jax.ShapeDtypeStruct((N_PAD, L), _f32)],
)


def _combine(s_ref, a0, a1, invd):
  t = s_ref[...] + (a0 + a1) * invd
  t = jnp.maximum(t, 0.0)
  nrm = jnp.sqrt(jnp.sum(t * t, axis=-1, keepdims=True))
  return t / jnp.maximum(nrm, 1e-12)


def _invdeg(a0, a1):
  return 1.0 / jnp.maximum(a0[:, H:H + 1] + a1[:, H:H + 1], 1.0)


def _tc2_body(s1, a0, a1, ws2, wn2, b2, s2, g2):
  invd = _invdeg(a0[...], a1[...])
  h1 = _combine(s1, a0[:, :H], a1[:, :H], invd)
  s2[...] = jnp.dot(h1, ws2[...], preferred_element_type=_f32) + b2[...]
  g2[...] = jnp.dot(h1, wn2[...], preferred_element_type=_f32)


_tc2 = pl.pallas_call(
    _tc2_body,
    grid=_GRID,
    in_specs=[_row_spec(H), _row_spec(AW), _row_spec(AW),
              _full_spec(H, H), _full_spec(H, H), _full_spec(1, H)],
    out_specs=[_row_spec(H), _row_spec(H)],
    out_shape=[jax.ShapeDtypeStruct((N_PAD, H), _f32),
               jax.ShapeDtypeStruct((N_PAD, H), _f32)],
)


def _tc3_body(s2, a0, a1, d0, d1, zm,
              wmu, bmu, wdec, bdec, wsc, bsc, wr, br, wdo, bdo,
              scale, r_out, do_out):
  invd = _invdeg(d0[...], d1[...])
  h2 = _combine(s2, a0[...], a1[...], invd)
  z_loc = jnp.dot(h2, wmu[...], preferred_element_type=_f32) + bmu[...]
  hz = z_loc * zm[...]
  px = jnp.maximum(
      jnp.dot(hz, wdec[...], preferred_element_type=_f32) + bdec[...], 0.0)
  logits = jnp.dot(px, wsc[...], preferred_element_type=_f32) + bsc[...]
  m = jnp.max(logits, axis=-1, keepdims=True)
  e = jnp.exp(logits - m)
  scale[...] = e / jnp.sum(e, axis=-1, keepdims=True)
  r_out[...] = jnp.dot(px, wr[...], preferred_element_type=_f32) + br[...]
  do_out[...] = jnp.dot(px, wdo[...], preferred_element_type=_f32) + bdo[...]


_tc3 = pl.pallas_call(
    _tc3_body,
    grid=_GRID,
    in_specs=[_row_spec(H), _row_spec(H), _row_spec(H),
              _row_spec(AW), _row_spec(AW), _row_spec(L),
              _full_spec(H, L), _full_spec(1, L),
              _full_spec(L, H), _full_spec(1, H),
              _full_spec(H, C), _full_spec(1, C),
              _full_spec(H, D), _full_spec(1, D),
              _full_spec(H, D), _full_spec(1, D)],
    out_specs=[_row_spec(C), _row_spec(D), _row_spec(D)],
    out_shape=[jax.ShapeDtypeStruct((N_PAD, C), _f32),
               jax.ShapeDtypeStruct((N_PAD, D), _f32),
               jax.ShapeDtypeStruct((N_PAD, D), _f32)],
)


def kernel(x, edge_index, W_self1, W_neigh1, b1, W_self2, W_neigh2, b2,
           W_mu, b_mu, W_fcm, b_fcm, W_mum, b_mum, W_dec, b_dec,
           W_sc, b_sc, W_r, b_r, W_do, b_do):
  src = edge_index[0].astype(jnp.int32)
  dst = edge_index[1].astype(jnp.int32)
  pad = E_PAD - E
  srcb = jnp.concatenate([src, jnp.zeros((pad,), jnp.int32)])
  dstb = jnp.concatenate([dst, jnp.full((pad,), N, jnp.int32)])
  srcb = srcb.reshape(E_PAD // LANE, LANE)
  dstb = dstb.reshape(E_PAD // LANE, LANE)

  xp = jnp.zeros((N_PAD, D), _f32).at[:N].set(x)
  r1 = lambda b: b.reshape(1, -1)

  s1, g1, zm = _tc1(xp, W_self1, W_neigh1, r1(b1), W_fcm, r1(b_fcm),
                    W_mum, r1(b_mum))
  acc1 = _get_spmm(AW)(g1, srcb, dstb)
  s2, g2 = _tc2(s1, acc1[0], acc1[1], W_self2, W_neigh2, r1(b2))
  acc2 = _get_spmm(H)(g2, srcb, dstb)
  px_scale, px_r, px_dropout = _tc3(
      s2, acc2[0], acc2[1], acc1[0], acc1[1], zm,
      W_mu, r1(b_mu), W_dec, r1(b_dec), W_sc, r1(b_sc),
      W_r, r1(b_r), W_do, r1(b_do))
  return (px_scale[:N], px_r[:N], px_dropout[:N])
